# baseline (device time: 416573 ns/iter reference)
import jax
import jax.numpy as jnp
from jax import lax
from jax.experimental import pallas as pl
from jax.experimental.pallas import tpu as pltpu

N_DEV = 32


def kernel(A, B):
    m, k = A.shape
    _, n = B.shape

    def body(a_ref, b_ref, out_ref, comm_ref, send_sems, recv_sems, credit_sem):
        my = lax.axis_index("i")
        left = lax.rem(my - 1 + N_DEV, N_DEV)
        right = lax.rem(my + 1, N_DEV)

        barrier_sem = pltpu.get_barrier_semaphore()
        for nbr in (left, right):
            pl.semaphore_signal(
                barrier_sem, inc=1,
                device_id=(nbr,), device_id_type=pl.DeviceIdType.MESH,
            )
        pl.semaphore_wait(barrier_sem, 2)

        part = jnp.dot(a_ref[:, :], b_ref[:, :],
                       preferred_element_type=jnp.float32)
        out_ref[:, :] = part
        comm_ref[0, :, :] = part

        n_hops = N_DEV - 1
        for h in range(n_hops):
            s = h % 2
            r = (h + 1) % 2
            if h >= 2:
                pl.semaphore_wait(credit_sem, 1)
            rdma = pltpu.make_async_remote_copy(
                src_ref=comm_ref.at[s],
                dst_ref=comm_ref.at[r],
                send_sem=send_sems.at[s],
                recv_sem=recv_sems.at[r],
                device_id=(right,),
                device_id_type=pl.DeviceIdType.MESH,
            )
            rdma.start()
            rdma.wait()
            out_ref[:, :] += comm_ref[r, :, :]
            if h <= n_hops - 3:
                pl.semaphore_signal(
                    credit_sem, inc=1,
                    device_id=(left,), device_id_type=pl.DeviceIdType.MESH,
                )

        z = out_ref[:, :]
        out_ref[:, :] = z * (1.0 / (1.0 + jnp.exp(-z)))

    return pl.pallas_call(
        body,
        out_shape=jax.ShapeDtypeStruct((m, n), jnp.float32),
        in_specs=[
            pl.BlockSpec(memory_space=pltpu.VMEM),
            pl.BlockSpec(memory_space=pltpu.VMEM),
        ],
        out_specs=pl.BlockSpec(memory_space=pltpu.VMEM),
        scratch_shapes=[
            pltpu.VMEM((2, m, n), jnp.float32),
            pltpu.SemaphoreType.DMA((2,)),
            pltpu.SemaphoreType.DMA((2,)),
            pltpu.SemaphoreType.REGULAR,
        ],
        compiler_params=pltpu.CompilerParams(collective_id=0),
    )(A, B)


# device time: 48649 ns/iter; 8.5628x vs baseline; 8.5628x over previous
import jax
import jax.numpy as jnp
from jax import lax
from jax.experimental import pallas as pl
from jax.experimental.pallas import tpu as pltpu

N_DEV = 32
MASKS = (8, 1, 2, 4, 16)
SLAB_OFF = (0, 256, 384, 448, 480)
SLAB_ROWS = 496


def kernel(A, B):
    m, k = A.shape
    _, n = B.shape

    def body(a_ref, b_ref, out_ref, comm_ref, send_sems, recv_sems):
        my = lax.axis_index("i")

        barrier_sem = pltpu.get_barrier_semaphore()
        for mask in MASKS:
            pl.semaphore_signal(
                barrier_sem, inc=1,
                device_id=(my ^ mask,), device_id_type=pl.DeviceIdType.MESH,
            )
        pl.semaphore_wait(barrier_sem, len(MASKS))

        out_ref[:, :] = jnp.dot(a_ref[:, :], b_ref[:, :],
                                preferred_element_type=jnp.float32)

        off = jnp.int32(0)
        L = m
        for rnd, mask in enumerate(MASKS):
            half = L // 2
            b = ((my & mask) != 0).astype(jnp.int32)
            keep_off = off + b * half
            send_off = off + (1 - b) * half
            rdma = pltpu.make_async_remote_copy(
                src_ref=out_ref.at[pl.ds(send_off, half), :],
                dst_ref=comm_ref.at[pl.ds(SLAB_OFF[rnd], half), :],
                send_sem=send_sems.at[rnd],
                recv_sem=recv_sems.at[rnd],
                device_id=(my ^ mask,),
                device_id_type=pl.DeviceIdType.MESH,
            )
            rdma.start()
            rdma.wait()
            s0 = SLAB_OFF[rnd]
            out_ref[pl.ds(keep_off, half), :] = (
                out_ref[pl.ds(keep_off, half), :]
                + comm_ref[pl.ds(s0, half), :]
            )
            off = keep_off
            L = half

        z = out_ref[pl.ds(off, L), :]
        out_ref[pl.ds(off, L), :] = z * (1.0 / (1.0 + jnp.exp(-z)))

        for rnd, mask in enumerate(reversed(MASKS)):
            b = ((my & mask) != 0).astype(jnp.int32)
            rdma = pltpu.make_async_remote_copy(
                src_ref=out_ref.at[pl.ds(off, L), :],
                dst_ref=out_ref.at[pl.ds(off, L), :],
                send_sem=send_sems.at[len(MASKS) + rnd],
                recv_sem=recv_sems.at[len(MASKS) + rnd],
                device_id=(my ^ mask,),
                device_id_type=pl.DeviceIdType.MESH,
            )
            rdma.start()
            rdma.wait()
            off = off - b * L
            L = 2 * L

    return pl.pallas_call(
        body,
        out_shape=jax.ShapeDtypeStruct((m, n), jnp.float32),
        in_specs=[
            pl.BlockSpec(memory_space=pltpu.VMEM),
            pl.BlockSpec(memory_space=pltpu.VMEM),
        ],
        out_specs=pl.BlockSpec(memory_space=pltpu.VMEM),
        scratch_shapes=[
            pltpu.VMEM((SLAB_ROWS, n), jnp.float32),
            pltpu.SemaphoreType.DMA((2 * len(MASKS),)),
            pltpu.SemaphoreType.DMA((2 * len(MASKS),)),
        ],
        compiler_params=pltpu.CompilerParams(collective_id=0),
    )(A, B)


# device time: 48497 ns/iter; 8.5897x vs baseline; 1.0031x over previous
import jax
import jax.numpy as jnp
from jax import lax
from jax.experimental import pallas as pl
from jax.experimental.pallas import tpu as pltpu

N_DEV = 32
MASKS = (8, 1, 2, 4, 16)
SLAB_OFF = (0, 256, 384, 448, 480)
SLAB_ROWS = 496


def kernel(A, B):
    m, k = A.shape
    _, n = B.shape

    def body(a_ref, b_ref, out_ref, comm_ref, send_sems, recv_sems):
        my = lax.axis_index("i")

        barrier_sem = pltpu.get_barrier_semaphore()
        for mask in MASKS:
            pl.semaphore_signal(
                barrier_sem, inc=1,
                device_id=(my ^ mask,), device_id_type=pl.DeviceIdType.MESH,
            )
        pl.semaphore_wait(barrier_sem, len(MASKS))

        pending = []

        b0 = ((my & MASKS[0]) != 0).astype(jnp.int32)
        half0 = m // 2
        keep0 = b0 * half0
        send0 = (1 - b0) * half0
        out_ref[pl.ds(send0, half0), :] = jnp.dot(
            a_ref[pl.ds(send0, half0), :], b_ref[:, :],
            preferred_element_type=jnp.float32)
        rdma0 = pltpu.make_async_remote_copy(
            src_ref=out_ref.at[pl.ds(send0, half0), :],
            dst_ref=comm_ref.at[pl.ds(SLAB_OFF[0], half0), :],
            send_sem=send_sems.at[0],
            recv_sem=recv_sems.at[0],
            device_id=(my ^ MASKS[0],),
            device_id_type=pl.DeviceIdType.MESH,
        )
        rdma0.start()
        pending.append(rdma0)
        out_ref[pl.ds(keep0, half0), :] = jnp.dot(
            a_ref[pl.ds(keep0, half0), :], b_ref[:, :],
            preferred_element_type=jnp.float32)
        rdma0.wait_recv()
        out_ref[pl.ds(keep0, half0), :] = (
            out_ref[pl.ds(keep0, half0), :]
            + comm_ref[pl.ds(SLAB_OFF[0], half0), :]
        )
        off = keep0
        L = half0

        for rnd, mask in enumerate(MASKS[1:], start=1):
            half = L // 2
            b = ((my & mask) != 0).astype(jnp.int32)
            keep_off = off + b * half
            send_off = off + (1 - b) * half
            rdma = pltpu.make_async_remote_copy(
                src_ref=out_ref.at[pl.ds(send_off, half), :],
                dst_ref=comm_ref.at[pl.ds(SLAB_OFF[rnd], half), :],
                send_sem=send_sems.at[rnd],
                recv_sem=recv_sems.at[rnd],
                device_id=(my ^ mask,),
                device_id_type=pl.DeviceIdType.MESH,
            )
            rdma.start()
            pending.append(rdma)
            rdma.wait_recv()
            s0 = SLAB_OFF[rnd]
            out_ref[pl.ds(keep_off, half), :] = (
                out_ref[pl.ds(keep_off, half), :]
                + comm_ref[pl.ds(s0, half), :]
            )
            off = keep_off
            L = half

        z = out_ref[pl.ds(off, L), :]
        out_ref[pl.ds(off, L), :] = z * (1.0 / (1.0 + jnp.exp(-z)))

        for rnd, mask in enumerate(reversed(MASKS)):
            b = ((my & mask) != 0).astype(jnp.int32)
            rdma = pltpu.make_async_remote_copy(
                src_ref=out_ref.at[pl.ds(off, L), :],
                dst_ref=out_ref.at[pl.ds(off, L), :],
                send_sem=send_sems.at[len(MASKS) + rnd],
                recv_sem=recv_sems.at[len(MASKS) + rnd],
                device_id=(my ^ mask,),
                device_id_type=pl.DeviceIdType.MESH,
            )
            rdma.start()
            pending.append(rdma)
            rdma.wait_recv()
            off = off - b * L
            L = 2 * L

        for rdma in pending:
            rdma.wait_send()

    return pl.pallas_call(
        body,
        out_shape=jax.ShapeDtypeStruct((m, n), jnp.float32),
        in_specs=[
            pl.BlockSpec(memory_space=pltpu.VMEM),
            pl.BlockSpec(memory_space=pltpu.VMEM),
        ],
        out_specs=pl.BlockSpec(memory_space=pltpu.VMEM),
        scratch_shapes=[
            pltpu.VMEM((SLAB_ROWS, n), jnp.float32),
            pltpu.SemaphoreType.DMA((2 * len(MASKS),)),
            pltpu.SemaphoreType.DMA((2 * len(MASKS),)),
        ],
        compiler_params=pltpu.CompilerParams(collective_id=0),
    )(A, B)


# device time: 43814 ns/iter; 9.5078x vs baseline; 1.1069x over previous
import jax
import jax.numpy as jnp
from jax import lax
from jax.experimental import pallas as pl
from jax.experimental.pallas import tpu as pltpu

N_DEV = 32
MASKS = (8, 1, 2, 4, 16)
N_R = len(MASKS)
SLAB_OFF = (0, 256, 384, 448, 480)
SLAB_ROWS = 496


def kernel(A, B):
    m, k = A.shape
    _, n = B.shape
    colw = n // 2

    def body(a_ref, b_ref, out_ref, comm_ref, send_sems, recv_sems):
        my = lax.axis_index("i")

        barrier_sem = pltpu.get_barrier_semaphore()
        for mask in MASKS:
            pl.semaphore_signal(
                barrier_sem, inc=1,
                device_id=(my ^ mask,), device_id_type=pl.DeviceIdType.MESH,
            )
        pl.semaphore_wait(barrier_sem, N_R)

        pending = []

        def bit(mask):
            return ((my & mask) != 0).astype(jnp.int32)

        def rs_issue(c, r, off, L):
            half = L // 2
            b = bit(MASKS[r])
            keep_off = off + b * half
            send_off = off + (1 - b) * half
            rdma = pltpu.make_async_remote_copy(
                src_ref=out_ref.at[pl.ds(send_off, half),
                                   pl.ds(c * colw, colw)],
                dst_ref=comm_ref.at[pl.ds(SLAB_OFF[r], half),
                                    pl.ds(c * colw, colw)],
                send_sem=send_sems.at[N_R * 2 * c + r],
                recv_sem=recv_sems.at[N_R * 2 * c + r],
                device_id=(my ^ MASKS[r],),
                device_id_type=pl.DeviceIdType.MESH,
            )
            rdma.start()
            pending.append(rdma)
            return rdma, keep_off, half

        def rs_complete(c, r, rdma, keep_off, half):
            rdma.wait_recv()
            out_ref[pl.ds(keep_off, half), pl.ds(c * colw, colw)] = (
                out_ref[pl.ds(keep_off, half), pl.ds(c * colw, colw)]
                + comm_ref[pl.ds(SLAB_OFF[r], half), pl.ds(c * colw, colw)]
            )

        def ag_issue(c, r, off, L):
            rdma = pltpu.make_async_remote_copy(
                src_ref=out_ref.at[pl.ds(off, L), pl.ds(c * colw, colw)],
                dst_ref=out_ref.at[pl.ds(off, L), pl.ds(c * colw, colw)],
                send_sem=send_sems.at[N_R * 2 * c + N_R + r],
                recv_sem=recv_sems.at[N_R * 2 * c + N_R + r],
                device_id=(my ^ MASKS[N_R - 1 - r],),
                device_id_type=pl.DeviceIdType.MESH,
            )
            rdma.start()
            pending.append(rdma)
            return rdma

        b0 = bit(MASKS[0])
        half0 = m // 2
        send0 = (1 - b0) * half0
        keep0 = b0 * half0
        out_ref[pl.ds(send0, half0), :] = jnp.dot(
            a_ref[pl.ds(send0, half0), :], b_ref[:, :],
            preferred_element_type=jnp.float32)
        st = {}
        st[0] = rs_issue(0, 0, jnp.int32(0), m)
        st[1] = rs_issue(1, 0, jnp.int32(0), m)
        out_ref[pl.ds(keep0, half0), :] = jnp.dot(
            a_ref[pl.ds(keep0, half0), :], b_ref[:, :],
            preferred_element_type=jnp.float32)

        offL = {0: (jnp.int32(0), m), 1: (jnp.int32(0), m)}
        for r in range(N_R):
            for c in (0, 1):
                rdma, keep_off, half = st[c]
                rs_complete(c, r, rdma, keep_off, half)
                offL[c] = (keep_off, half)
                if r + 1 < N_R:
                    st[c] = rs_issue(c, r + 1, keep_off, half)

        off, L = offL[0]
        z = out_ref[pl.ds(off, L), :]
        out_ref[pl.ds(off, L), :] = z * (1.0 / (1.0 + jnp.exp(-z)))

        st = {0: ag_issue(0, 0, off, L), 1: ag_issue(1, 0, off, L)}
        for r in range(N_R):
            mask = MASKS[N_R - 1 - r]
            b = bit(mask)
            for c in (0, 1):
                st[c].wait_recv()
                o, Lc = offL[c]
                offL[c] = (o - b * Lc, 2 * Lc)
                if r + 1 < N_R:
                    st[c] = ag_issue(c, r + 1, *offL[c])

        for rdma in pending:
            rdma.wait_send()

    return pl.pallas_call(
        body,
        out_shape=jax.ShapeDtypeStruct((m, n), jnp.float32),
        in_specs=[
            pl.BlockSpec(memory_space=pltpu.VMEM),
            pl.BlockSpec(memory_space=pltpu.VMEM),
        ],
        out_specs=pl.BlockSpec(memory_space=pltpu.VMEM),
        scratch_shapes=[
            pltpu.VMEM((SLAB_ROWS, n), jnp.float32),
            pltpu.SemaphoreType.DMA((4 * N_R,)),
            pltpu.SemaphoreType.DMA((4 * N_R,)),
        ],
        compiler_params=pltpu.CompilerParams(collective_id=0),
    )(A, B)


# device time: 41551 ns/iter; 10.0256x vs baseline; 1.0545x over previous
import jax
import jax.numpy as jnp
from jax import lax
from jax.experimental import pallas as pl
from jax.experimental.pallas import tpu as pltpu

N_DEV = 32
MASKS = (8, 1, 2, 4, 16)
N_R = len(MASKS)
NC = 4
SLAB_OFF = (0, 256, 384, 448, 480)
SLAB_ROWS = 496


def kernel(A, B):
    m, k = A.shape
    _, n = B.shape
    colw = n // NC

    def body(a_ref, b_ref, out_ref, comm_ref, send_sems, recv_sems):
        my = lax.axis_index("i")

        barrier_sem = pltpu.get_barrier_semaphore()
        for mask in MASKS:
            pl.semaphore_signal(
                barrier_sem, inc=1,
                device_id=(my ^ mask,), device_id_type=pl.DeviceIdType.MESH,
            )
        pl.semaphore_wait(barrier_sem, N_R)

        pending = []

        def bit(mask):
            return ((my & mask) != 0).astype(jnp.int32)

        def rs_issue(c, r, off, L):
            half = L // 2
            b = bit(MASKS[r])
            keep_off = off + b * half
            send_off = off + (1 - b) * half
            rdma = pltpu.make_async_remote_copy(
                src_ref=out_ref.at[pl.ds(send_off, half),
                                   pl.ds(c * colw, colw)],
                dst_ref=comm_ref.at[pl.ds(SLAB_OFF[r], half),
                                    pl.ds(c * colw, colw)],
                send_sem=send_sems.at[2 * N_R * c + r],
                recv_sem=recv_sems.at[2 * N_R * c + r],
                device_id=(my ^ MASKS[r],),
                device_id_type=pl.DeviceIdType.MESH,
            )
            rdma.start()
            pending.append(rdma)
            return rdma, keep_off, half

        def rs_complete(c, r, rdma, keep_off, half):
            rdma.wait_recv()
            out_ref[pl.ds(keep_off, half), pl.ds(c * colw, colw)] = (
                out_ref[pl.ds(keep_off, half), pl.ds(c * colw, colw)]
                + comm_ref[pl.ds(SLAB_OFF[r], half), pl.ds(c * colw, colw)]
            )

        def ag_issue(c, r, off, L):
            rdma = pltpu.make_async_remote_copy(
                src_ref=out_ref.at[pl.ds(off, L), pl.ds(c * colw, colw)],
                dst_ref=out_ref.at[pl.ds(off, L), pl.ds(c * colw, colw)],
                send_sem=send_sems.at[2 * N_R * c + N_R + r],
                recv_sem=recv_sems.at[2 * N_R * c + N_R + r],
                device_id=(my ^ MASKS[N_R - 1 - r],),
                device_id_type=pl.DeviceIdType.MESH,
            )
            rdma.start()
            pending.append(rdma)
            return rdma

        b0 = bit(MASKS[0])
        half0 = m // 2
        send0 = (1 - b0) * half0
        keep0 = b0 * half0
        out_ref[pl.ds(send0, half0), :] = jnp.dot(
            a_ref[pl.ds(send0, half0), :], b_ref[:, :],
            preferred_element_type=jnp.float32)
        st = {}
        for c in range(NC):
            st[c] = rs_issue(c, 0, jnp.int32(0), m)
        out_ref[pl.ds(keep0, half0), :] = jnp.dot(
            a_ref[pl.ds(keep0, half0), :], b_ref[:, :],
            preferred_element_type=jnp.float32)

        offL = {c: (jnp.int32(0), m) for c in range(NC)}
        for r in range(N_R):
            for c in range(NC):
                rdma, keep_off, half = st[c]
                rs_complete(c, r, rdma, keep_off, half)
                offL[c] = (keep_off, half)
                if r + 1 < N_R:
                    st[c] = rs_issue(c, r + 1, keep_off, half)

        off, L = offL[0]
        z = out_ref[pl.ds(off, L), :]
        out_ref[pl.ds(off, L), :] = z * (1.0 / (1.0 + jnp.exp(-z)))

        st = {c: ag_issue(c, 0, off, L) for c in range(NC)}
        for r in range(N_R):
            mask = MASKS[N_R - 1 - r]
            b = bit(mask)
            for c in range(NC):
                st[c].wait_recv()
                o, Lc = offL[c]
                offL[c] = (o - b * Lc, 2 * Lc)
                if r + 1 < N_R:
                    st[c] = ag_issue(c, r + 1, *offL[c])

        for rdma in pending:
            rdma.wait_send()

    return pl.pallas_call(
        body,
        out_shape=jax.ShapeDtypeStruct((m, n), jnp.float32),
        in_specs=[
            pl.BlockSpec(memory_space=pltpu.VMEM),
            pl.BlockSpec(memory_space=pltpu.VMEM),
        ],
        out_specs=pl.BlockSpec(memory_space=pltpu.VMEM),
        scratch_shapes=[
            pltpu.VMEM((SLAB_ROWS, n), jnp.float32),
            pltpu.SemaphoreType.DMA((2 * N_R * NC,)),
            pltpu.SemaphoreType.DMA((2 * N_R * NC,)),
        ],
        compiler_params=pltpu.CompilerParams(collective_id=0),
    )(A, B)


# device time: 40879 ns/iter; 10.1904x vs baseline; 1.0164x over previous
import jax
import jax.numpy as jnp
from jax import lax
from jax.experimental import pallas as pl
from jax.experimental.pallas import tpu as pltpu

N_DEV = 32
MASKS = (8, 1, 2, 4, 16)
N_R = len(MASKS)
NC = 4
SLAB_OFF = (0, 256, 384, 448, 480)
SLAB_ROWS = 496


def kernel(A, B):
    m, k = A.shape
    _, n = B.shape
    colw = n // NC

    def body(a_ref, b_ref, out_ref, comm_ref, send_sems, recv_sems):
        my = lax.axis_index("i")

        barrier_sem = pltpu.get_barrier_semaphore()
        for mask in MASKS:
            pl.semaphore_signal(
                barrier_sem, inc=1,
                device_id=(my ^ mask,), device_id_type=pl.DeviceIdType.MESH,
            )
        pl.semaphore_wait(barrier_sem, N_R)

        pending = []

        def bit(mask):
            return ((my & mask) != 0).astype(jnp.int32)

        def rs_issue(c, r, off, L):
            half = L // 2
            b = bit(MASKS[r])
            keep_off = off + b * half
            send_off = off + (1 - b) * half
            rdma = pltpu.make_async_remote_copy(
                src_ref=out_ref.at[pl.ds(send_off, half),
                                   pl.ds(c * colw, colw)],
                dst_ref=comm_ref.at[pl.ds(SLAB_OFF[r], half),
                                    pl.ds(c * colw, colw)],
                send_sem=send_sems.at[2 * N_R * c + r],
                recv_sem=recv_sems.at[2 * N_R * c + r],
                device_id=(my ^ MASKS[r],),
                device_id_type=pl.DeviceIdType.MESH,
            )
            rdma.start()
            pending.append(rdma)
            return rdma, keep_off, half

        def rs_complete(c, r, rdma, keep_off, half):
            rdma.wait_recv()
            out_ref[pl.ds(keep_off, half), pl.ds(c * colw, colw)] = (
                out_ref[pl.ds(keep_off, half), pl.ds(c * colw, colw)]
                + comm_ref[pl.ds(SLAB_OFF[r], half), pl.ds(c * colw, colw)]
            )

        def ag_issue(c, r, off, L):
            rdma = pltpu.make_async_remote_copy(
                src_ref=out_ref.at[pl.ds(off, L), pl.ds(c * colw, colw)],
                dst_ref=out_ref.at[pl.ds(off, L), pl.ds(c * colw, colw)],
                send_sem=send_sems.at[2 * N_R * c + N_R + r],
                recv_sem=recv_sems.at[2 * N_R * c + N_R + r],
                device_id=(my ^ MASKS[N_R - 1 - r],),
                device_id_type=pl.DeviceIdType.MESH,
            )
            rdma.start()
            pending.append(rdma)
            return rdma

        b0 = bit(MASKS[0])
        half0 = m // 2
        send0 = (1 - b0) * half0
        keep0 = b0 * half0
        out_ref[pl.ds(send0, half0), :] = jnp.dot(
            a_ref[pl.ds(send0, half0), :], b_ref[:, :],
            preferred_element_type=jnp.float32)
        st = {}
        for c in range(NC):
            st[c] = rs_issue(c, 0, jnp.int32(0), m)
        out_ref[pl.ds(keep0, half0), :] = jnp.dot(
            a_ref[pl.ds(keep0, half0), :], b_ref[:, :],
            preferred_element_type=jnp.float32)

        offL = {c: (jnp.int32(0), m) for c in range(NC)}
        ag_st = {}
        for r in range(N_R):
            for c in range(NC):
                rdma, keep_off, half = st[c]
                rs_complete(c, r, rdma, keep_off, half)
                offL[c] = (keep_off, half)
                if r + 1 < N_R:
                    st[c] = rs_issue(c, r + 1, keep_off, half)
                else:
                    z = out_ref[pl.ds(keep_off, half), pl.ds(c * colw, colw)]
                    out_ref[pl.ds(keep_off, half), pl.ds(c * colw, colw)] = (
                        z * (1.0 / (1.0 + jnp.exp(-z))))
                    ag_st[c] = ag_issue(c, 0, keep_off, half)

        st = ag_st
        for r in range(N_R):
            mask = MASKS[N_R - 1 - r]
            b = bit(mask)
            for c in range(NC):
                st[c].wait_recv()
                o, Lc = offL[c]
                offL[c] = (o - b * Lc, 2 * Lc)
                if r + 1 < N_R:
                    st[c] = ag_issue(c, r + 1, *offL[c])

        for rdma in pending:
            rdma.wait_send()

    return pl.pallas_call(
        body,
        out_shape=jax.ShapeDtypeStruct((m, n), jnp.float32),
        in_specs=[
            pl.BlockSpec(memory_space=pltpu.VMEM),
            pl.BlockSpec(memory_space=pltpu.VMEM),
        ],
        out_specs=pl.BlockSpec(memory_space=pltpu.VMEM),
        scratch_shapes=[
            pltpu.VMEM((SLAB_ROWS, n), jnp.float32),
            pltpu.SemaphoreType.DMA((2 * N_R * NC,)),
            pltpu.SemaphoreType.DMA((2 * N_R * NC,)),
        ],
        compiler_params=pltpu.CompilerParams(collective_id=0),
    )(A, B)


# device time: 40100 ns/iter; 10.3884x vs baseline; 1.0194x over previous
import jax
import jax.numpy as jnp
from jax import lax
from jax.experimental import pallas as pl
from jax.experimental.pallas import tpu as pltpu

N_DEV = 32
MASKS = (8, 1, 2, 4, 16)
N_R = len(MASKS)
NC = 4
SLAB_OFF = (0, 256, 384, 448, 480)
SLAB_ROWS = 496


def kernel(A, B):
    m, k = A.shape
    _, n = B.shape
    colw = n // NC

    def body(a_ref, b_ref, out_ref, comm_ref, send_sems, recv_sems):
        my = lax.axis_index("i")

        barrier_sem = pltpu.get_barrier_semaphore()
        for mask in MASKS:
            pl.semaphore_signal(
                barrier_sem, inc=1,
                device_id=(my ^ mask,), device_id_type=pl.DeviceIdType.MESH,
            )
        pl.semaphore_wait(barrier_sem, N_R)

        pending = []

        def bit(mask):
            return ((my & mask) != 0).astype(jnp.int32)

        SEMS_PER_CHAIN = 16

        def _copy(c, src_off, dst_off, rows, sem_idx, mask):
            rdma = pltpu.make_async_remote_copy(
                src_ref=out_ref.at[pl.ds(src_off, rows),
                                   pl.ds(c * colw, colw)],
                dst_ref=comm_ref.at[pl.ds(dst_off, rows),
                                    pl.ds(c * colw, colw)],
                send_sem=send_sems.at[SEMS_PER_CHAIN * c + sem_idx],
                recv_sem=recv_sems.at[SEMS_PER_CHAIN * c + sem_idx],
                device_id=(my ^ mask,),
                device_id_type=pl.DeviceIdType.MESH,
            )
            rdma.start()
            pending.append(rdma)
            return rdma

        def rs_issue(c, r, off, L):
            half = L // 2
            b = bit(MASKS[r])
            keep_off = off + b * half
            send_off = off + (1 - b) * half
            if r + 1 < N_R:
                q = half // 2
                bn = bit(MASKS[r + 1])
                u = (1 - bn) * q
                lz = bn * q
                rd_u = _copy(c, send_off + u, SLAB_OFF[r] + u, q,
                             2 * r, MASKS[r])
                rd_l = _copy(c, send_off + lz, SLAB_OFF[r] + lz, q,
                             2 * r + 1, MASKS[r])
                return rd_u, rd_l, keep_off, half, u, lz, q
            rd = _copy(c, send_off, SLAB_OFF[r], half, 2 * r, MASKS[r])
            return rd, None, keep_off, half, None, None, half

        def add_slab(c, r, out_off, slab_off, rows):
            out_ref[pl.ds(out_off, rows), pl.ds(c * colw, colw)] = (
                out_ref[pl.ds(out_off, rows), pl.ds(c * colw, colw)]
                + comm_ref[pl.ds(slab_off, rows), pl.ds(c * colw, colw)]
            )

        def ag_issue(c, r, off, L):
            rdma = pltpu.make_async_remote_copy(
                src_ref=out_ref.at[pl.ds(off, L), pl.ds(c * colw, colw)],
                dst_ref=out_ref.at[pl.ds(off, L), pl.ds(c * colw, colw)],
                send_sem=send_sems.at[SEMS_PER_CHAIN * c + 9 + r],
                recv_sem=recv_sems.at[SEMS_PER_CHAIN * c + 9 + r],
                device_id=(my ^ MASKS[N_R - 1 - r],),
                device_id_type=pl.DeviceIdType.MESH,
            )
            rdma.start()
            pending.append(rdma)
            return rdma

        b0 = bit(MASKS[0])
        half0 = m // 2
        send0 = (1 - b0) * half0
        keep0 = b0 * half0
        out_ref[pl.ds(send0, half0), :] = jnp.dot(
            a_ref[pl.ds(send0, half0), :], b_ref[:, :],
            preferred_element_type=jnp.float32)
        st = {}
        for c in range(NC):
            st[c] = rs_issue(c, 0, jnp.int32(0), m)
        out_ref[pl.ds(keep0, half0), :] = jnp.dot(
            a_ref[pl.ds(keep0, half0), :], b_ref[:, :],
            preferred_element_type=jnp.float32)

        offL = {c: (jnp.int32(0), m) for c in range(NC)}
        ag_st = {}
        for r in range(N_R):
            for c in range(NC):
                rd_u, rd_l, keep_off, half, u, lz, q = st[c]
                rd_u.wait_recv()
                add_slab(c, r, keep_off + (u if rd_l is not None else 0),
                         SLAB_OFF[r] + (u if rd_l is not None else 0), q)
                offL[c] = (keep_off, half)
                if r + 1 < N_R:
                    st[c] = rs_issue(c, r + 1, keep_off, half)
                    rd_l.wait_recv()
                    add_slab(c, r, keep_off + lz, SLAB_OFF[r] + lz, q)
                else:
                    z = out_ref[pl.ds(keep_off, half), pl.ds(c * colw, colw)]
                    out_ref[pl.ds(keep_off, half), pl.ds(c * colw, colw)] = (
                        z * (1.0 / (1.0 + jnp.exp(-z))))
                    ag_st[c] = ag_issue(c, 0, keep_off, half)

        st = ag_st
        for r in range(N_R):
            mask = MASKS[N_R - 1 - r]
            b = bit(mask)
            for c in range(NC):
                st[c].wait_recv()
                o, Lc = offL[c]
                offL[c] = (o - b * Lc, 2 * Lc)
                if r + 1 < N_R:
                    st[c] = ag_issue(c, r + 1, *offL[c])

        for rdma in pending:
            rdma.wait_send()

    return pl.pallas_call(
        body,
        out_shape=jax.ShapeDtypeStruct((m, n), jnp.float32),
        in_specs=[
            pl.BlockSpec(memory_space=pltpu.VMEM),
            pl.BlockSpec(memory_space=pltpu.VMEM),
        ],
        out_specs=pl.BlockSpec(memory_space=pltpu.VMEM),
        scratch_shapes=[
            pltpu.VMEM((SLAB_ROWS, n), jnp.float32),
            pltpu.SemaphoreType.DMA((16 * NC,)),
            pltpu.SemaphoreType.DMA((16 * NC,)),
        ],
        compiler_params=pltpu.CompilerParams(collective_id=0),
    )(A, B)
